# Initial kernel scaffold; baseline (speedup 1.0000x reference)
import jax
import jax.numpy as jnp
from jax.experimental import pallas as pl


def kernel(edge_index_u2a, edge_index_a2u, emb_user, emb_app, W1_self_u2a, W1_neigh_u2a, b1_u2a, W1_self_a2u, W1_neigh_a2u, b1_a2u, W2_self_u2a, W2_neigh_u2a, b2_u2a, W_cls, b_cls):
    raise NotImplementedError("write your pallas kernel here")


# speedup vs baseline: 2.6054x; 2.6054x over previous
import functools

import jax
import jax.numpy as jnp
from jax import lax
from jax.experimental import pallas as pl
from jax.experimental.pallas import tpu as pltpu
from jax.experimental.pallas import tpu_sc as plsc

N_USER = 50000
N_APP = 50000
E = 300000
IN = 128
HID = 128
OUT = 16

NC = 2
NS = 16
NW = NC * NS
CH = 128
K = 74
E_PAD = NW * K * CH
NPAD = 50176
STRIPE = NPAD // NS
ZCH = 64
NZ = STRIPE // ZCH

_mesh = plsc.VectorSubcoreMesh(core_axis_name="c", subcore_axis_name="s")


def _sc_degree(dst_u, dst_a, ones_hbm, zb_hbm):
    @functools.partial(
        pl.kernel,
        out_type=jax.ShapeDtypeStruct((2, 2, NPAD, 16), jnp.float32),
        mesh=_mesh,
        compiler_params=pltpu.CompilerParams(use_tc_tiling_on_sc=False),
        scratch_types=[
            pltpu.VMEM((K, CH), jnp.int32),
            pltpu.VMEM((CH, 16), jnp.float32),
            pltpu.VMEM((ZCH, 16), jnp.float32),
            pltpu.VMEM_SHARED((NPAD, 16), jnp.float32),
        ],
    )
    def deg_kernel(du_h, da_h, ones_h, zb_h, out_h, idx_v, ones_v, zv, acc):
        c = lax.axis_index("c")
        s = lax.axis_index("s")
        wid = c * NS + s
        tb = s * STRIPE
        pltpu.sync_copy(ones_h, ones_v)
        pltpu.sync_copy(zb_h, zv)
        for rel in range(2):
            @pl.loop(0, NZ)
            def _(j):
                pltpu.sync_copy(zv, acc.at[pl.ds(tb + j * ZCH, ZCH)])

            plsc.subcore_barrier()
            pltpu.sync_copy((du_h if rel == 0 else da_h).at[wid], idx_v)

            @pl.loop(0, K)
            def _(j):
                pltpu.sync_copy(ones_v, acc.at[idx_v.at[j]], add=True)

            plsc.subcore_barrier()
            sl = pl.ds(tb, STRIPE)
            pltpu.sync_copy(acc.at[sl], out_h.at[rel, c, sl])
            plsc.subcore_barrier()

    return deg_kernel(dst_u, dst_a, ones_hbm, zb_hbm)


NB = 8
BW = 16


def _sc_agg128(src8, dst3, table, zb_hbm):
    @functools.partial(
        pl.kernel,
        out_type=jax.ShapeDtypeStruct((NB, 2, NPAD, BW), jnp.float32),
        mesh=_mesh,
        compiler_params=pltpu.CompilerParams(use_tc_tiling_on_sc=False),
        scratch_types=[
            pltpu.VMEM((K, CH), jnp.int32),
            pltpu.VMEM((K, CH), jnp.int32),
            pltpu.VMEM((CH, BW), jnp.float32),
            pltpu.VMEM((ZCH, BW), jnp.float32),
            pltpu.VMEM_SHARED((NPAD, BW), jnp.float32),
            pltpu.SemaphoreType.DMA,
        ],
    )
    def agg_kernel(src_h, dst_h, table_h, zb_h, out_h, si_v, di_v, rows_v, zv,
                   acc, sem):
        c = lax.axis_index("c")
        s = lax.axis_index("s")
        wid = c * NS + s
        tb = s * STRIPE
        pltpu.sync_copy(zb_h, zv)
        pltpu.sync_copy(dst_h.at[wid], di_v)
        for b in range(NB):
            pltpu.sync_copy(src_h.at[b, wid], si_v)

            @pl.loop(0, NZ)
            def _(j):
                pltpu.sync_copy(zv, acc.at[pl.ds(tb + j * ZCH, ZCH)])

            plsc.subcore_barrier()

            @pl.loop(0, K)
            def _(j):
                pltpu.async_copy(table_h.at[si_v.at[j]], rows_v, sem).wait()
                pltpu.sync_copy(rows_v, acc.at[di_v.at[j]], add=True)

            plsc.subcore_barrier()
            pltpu.sync_copy(acc.at[pl.ds(tb, STRIPE)],
                            out_h.at[b, c, pl.ds(tb, STRIPE)])
            plsc.subcore_barrier()

    return agg_kernel(src8, dst3, table, zb_hbm)


def _sc_agg16(src3, dst3, table, zb_hbm):
    @functools.partial(
        pl.kernel,
        out_type=jax.ShapeDtypeStruct((2, NPAD, 16), jnp.float32),
        mesh=_mesh,
        compiler_params=pltpu.CompilerParams(use_tc_tiling_on_sc=False),
        scratch_types=[
            pltpu.VMEM((K, CH), jnp.int32),
            pltpu.VMEM((K, CH), jnp.int32),
            pltpu.VMEM((CH, 16), jnp.float32),
            pltpu.VMEM((ZCH, 16), jnp.float32),
            pltpu.VMEM_SHARED((NPAD, 16), jnp.float32),
            pltpu.SemaphoreType.DMA,
        ],
    )
    def agg_kernel(src_h, dst_h, table_h, zb_h, out_h, si_v, di_v, rows_v, zv,
                   acc, sem):
        c = lax.axis_index("c")
        s = lax.axis_index("s")
        wid = c * NS + s
        tb = s * STRIPE
        pltpu.sync_copy(zb_h, zv)
        pltpu.sync_copy(dst_h.at[wid], di_v)
        pltpu.sync_copy(src_h.at[wid], si_v)

        @pl.loop(0, NZ)
        def _(j):
            pltpu.sync_copy(zv, acc.at[pl.ds(tb + j * ZCH, ZCH)])

        plsc.subcore_barrier()

        @pl.loop(0, K)
        def _(j):
            pltpu.async_copy(table_h.at[si_v.at[j]], rows_v, sem).wait()
            pltpu.sync_copy(rows_v, acc.at[di_v.at[j]], add=True)

        plsc.subcore_barrier()
        sl = pl.ds(tb, STRIPE)
        pltpu.sync_copy(acc.at[sl], out_h.at[c, sl])

    return agg_kernel(src3, dst3, table, zb_hbm)


_BLK = 256


def _tc_sage(x, parts, d0, d1, Ws, Wn, bias, M):
    def body(x_ref, p_ref, d0_ref, d1_ref, ws_ref, wn_ref, b_ref,
             m_ref, o_ref):
        inv = 1.0 / jnp.maximum(d0_ref[:, 0:1] + d1_ref[:, 0:1], 1.0)
        h = jnp.dot(x_ref[...], ws_ref[...],
                    preferred_element_type=jnp.float32,
                    precision=lax.Precision.HIGHEST) + b_ref[...]
        wn = wn_ref[...]
        p = p_ref[...]
        for b in range(NB):
            hb = (p[b, 0] + p[b, 1]) * inv
            h = h + jnp.dot(hb, wn[BW * b:BW * (b + 1), :],
                            preferred_element_type=jnp.float32,
                            precision=lax.Precision.HIGHEST)
        h = jnp.maximum(h, 0.0)
        o_ref[...] = jnp.dot(h, m_ref[...], preferred_element_type=jnp.float32,
                             precision=lax.Precision.HIGHEST)

    return pl.pallas_call(
        body,
        grid=(NPAD // _BLK,),
        in_specs=[
            pl.BlockSpec((_BLK, 128), lambda i: (i, 0)),
            pl.BlockSpec((NB, 2, _BLK, BW), lambda i: (0, 0, i, 0)),
            pl.BlockSpec((_BLK, 16), lambda i: (i, 0)),
            pl.BlockSpec((_BLK, 16), lambda i: (i, 0)),
            pl.BlockSpec((128, 128), lambda i: (0, 0)),
            pl.BlockSpec((128, 128), lambda i: (0, 0)),
            pl.BlockSpec((1, 128), lambda i: (0, 0)),
            pl.BlockSpec((128, 16), lambda i: (0, 0)),
        ],
        out_specs=pl.BlockSpec((_BLK, 16), lambda i: (i, 0)),
        out_shape=jax.ShapeDtypeStruct((NPAD, 16), jnp.float32),
    )(x, parts, d0, d1, Ws, Wn, bias, M)


def _tc_final(happ, p0, p1, d0, d1, cvec):
    def body(h_ref, p0_ref, p1_ref, d0_ref, d1_ref, c_ref, o_ref):
        deg = d0_ref[:, 0:1] + d1_ref[:, 0:1]
        agg = (p0_ref[...] + p1_ref[...]) / jnp.maximum(deg, 1.0)
        o_ref[...] = h_ref[...] + agg + c_ref[...]

    return pl.pallas_call(
        body,
        grid=(NPAD // 1024,),
        in_specs=[
            pl.BlockSpec((1024, 16), lambda i: (i, 0)),
            pl.BlockSpec((1024, 16), lambda i: (i, 0)),
            pl.BlockSpec((1024, 16), lambda i: (i, 0)),
            pl.BlockSpec((1024, 16), lambda i: (i, 0)),
            pl.BlockSpec((1024, 16), lambda i: (i, 0)),
            pl.BlockSpec((1, 16), lambda i: (0, 0)),
        ],
        out_specs=pl.BlockSpec((1024, 16), lambda i: (i, 0)),
        out_shape=jax.ShapeDtypeStruct((NPAD, 16), jnp.float32),
    )(happ, p0, p1, d0, d1, cvec)


def kernel(edge_index_u2a, edge_index_a2u, emb_user, emb_app,
           W1_self_u2a, W1_neigh_u2a, b1_u2a,
           W1_self_a2u, W1_neigh_a2u, b1_a2u,
           W2_self_u2a, W2_neigh_u2a, b2_u2a,
           W_cls, b_cls):
    su = edge_index_u2a[0].astype(jnp.int32)
    du = edge_index_u2a[1].astype(jnp.int32)
    sa = edge_index_a2u[0].astype(jnp.int32)
    da = edge_index_a2u[1].astype(jnp.int32)

    pad = E_PAD - E
    su_p = jnp.concatenate([su, jnp.zeros((pad,), jnp.int32)])
    du_p = jnp.concatenate([du, jnp.full((pad,), N_APP, jnp.int32)])
    sa_p = jnp.concatenate([sa, jnp.zeros((pad,), jnp.int32)])
    da_p = jnp.concatenate([da, jnp.full((pad,), N_USER, jnp.int32)])

    dst_u3 = du_p.reshape(NW, K, CH)
    dst_a3 = da_p.reshape(NW, K, CH)
    src_u3 = su_p.reshape(NW, K, CH)
    su8 = jnp.stack([su_p * NB + b for b in range(NB)]).reshape(NB, NW, K, CH)
    sa8 = jnp.stack([sa_p * NB + b for b in range(NB)]).reshape(NB, NW, K, CH)

    ones16 = jnp.ones((CH, 16), jnp.float32)
    zb16 = jnp.zeros((ZCH, 16), jnp.float32)

    table_u = emb_user.reshape(N_USER * NB, BW)
    table_a = emb_app.reshape(N_APP * NB, BW)

    deg = _sc_degree(dst_u3, dst_a3, ones16, zb16)
    parts_app = _sc_agg128(su8, dst_u3, table_u, zb16)
    parts_user = _sc_agg128(sa8, dst_a3, table_a, zb16)

    A = W2_self_u2a @ W_cls
    Bm = W2_neigh_u2a @ W_cls
    cvec = (b2_u2a @ W_cls + b_cls).reshape(1, OUT)

    xu = jnp.pad(emb_user, ((0, NPAD - N_USER), (0, 0)))
    xa = jnp.pad(emb_app, ((0, NPAD - N_APP), (0, 0)))

    z_user = _tc_sage(xu, parts_user, deg[1, 0], deg[1, 1],
                      W1_self_a2u, W1_neigh_a2u, b1_a2u.reshape(1, HID), Bm)
    happ = _tc_sage(xa, parts_app, deg[0, 0], deg[0, 1],
                    W1_self_u2a, W1_neigh_u2a, b1_u2a.reshape(1, HID), A)

    parts_c = _sc_agg16(src_u3, dst_u3, z_user, zb16)

    out = _tc_final(happ, parts_c[0], parts_c[1], deg[0, 0], deg[0, 1], cvec)
    return out[:N_APP]


# speedup vs baseline: 3.4586x; 1.3275x over previous
import functools

import jax
import jax.numpy as jnp
from jax import lax
from jax.experimental import pallas as pl
from jax.experimental.pallas import tpu as pltpu
from jax.experimental.pallas import tpu_sc as plsc

N_USER = 50000
N_APP = 50000
E = 300000
IN = 128
HID = 128
OUT = 16

NC = 2
NS = 16
NW = NC * NS
CH = 128
K = 80
E_PAD = NW * K * CH
NPAD = 50176
STRIPE = NPAD // NS
FB = 4
ZCH = 196
NZB = STRIPE // ZCH

_mesh = plsc.VectorSubcoreMesh(core_axis_name="c", subcore_axis_name="s")


def _sc_degree(dst_u, dst_a, ones_hbm, zb_hbm):
    @functools.partial(
        pl.kernel,
        out_type=jax.ShapeDtypeStruct((2, 2, NPAD, 16), jnp.float32),
        mesh=_mesh,
        compiler_params=pltpu.CompilerParams(use_tc_tiling_on_sc=False),
        scratch_types=[
            pltpu.VMEM((K, CH), jnp.int32),
            pltpu.VMEM((CH, 16), jnp.float32),
            pltpu.VMEM((STRIPE, 16), jnp.float32),
            pltpu.VMEM_SHARED((NPAD, 16), jnp.float32),
        ],
    )
    def deg_kernel(du_h, da_h, ones_h, zb_h, out_h, idx_v, ones_v, zv, acc):
        c = lax.axis_index("c")
        s = lax.axis_index("s")
        wid = c * NS + s
        tb = s * STRIPE
        pltpu.sync_copy(ones_h, ones_v)
        pltpu.sync_copy(zb_h, zv)
        for rel in range(2):
            pltpu.sync_copy(zv, acc.at[pl.ds(tb, STRIPE)])
            plsc.subcore_barrier()
            pltpu.sync_copy((du_h if rel == 0 else da_h).at[wid], idx_v)

            @pl.loop(0, K)
            def _(j):
                pltpu.sync_copy(ones_v, acc.at[idx_v.at[j]], add=True)

            plsc.subcore_barrier()
            sl = pl.ds(tb, STRIPE)
            pltpu.sync_copy(acc.at[sl], out_h.at[rel, c, sl])
            plsc.subcore_barrier()

    return deg_kernel(dst_u, dst_a, ones_hbm, zb_hbm)


NB = 8
BW = 16


def _sc_agg128(src8, dst3, table, zb_hbm):
    @functools.partial(
        pl.kernel,
        out_type=jax.ShapeDtypeStruct((NB, 2, NPAD, BW), jnp.float32),
        mesh=_mesh,
        compiler_params=pltpu.CompilerParams(use_tc_tiling_on_sc=False),
        scratch_types=[
            pltpu.VMEM((K, CH), jnp.int32),
            pltpu.VMEM((K, CH), jnp.int32),
            [pltpu.VMEM((CH, BW), jnp.float32) for _ in range(2 * FB)],
            pltpu.VMEM((ZCH, BW), jnp.float32),
            pltpu.VMEM_SHARED((NPAD, BW), jnp.float32),
            pltpu.SemaphoreType.DMA,
            pltpu.SemaphoreType.DMA,
            pltpu.SemaphoreType.DMA,
            pltpu.SemaphoreType.DMA,
        ],
    )
    def agg_kernel(src_h, dst_h, table_h, zb_h, out_h, si_v, di_v, bufs, zv,
                   acc, sga, sgb, ssa, ssb):
        c = lax.axis_index("c")
        s = lax.axis_index("s")
        wid = c * NS + s
        tb = s * STRIPE
        pltpu.sync_copy(zb_h, zv)
        pltpu.sync_copy(dst_h.at[wid], di_v)
        for b in range(NB):
            pltpu.sync_copy(src_h.at[b, wid], si_v)
            zd = [pltpu.async_copy(zv, acc.at[pl.ds(tb + j * ZCH, ZCH)], sga)
                  for j in range(NZB)]
            for d in zd:
                d.wait()
            plsc.subcore_barrier()

            @pl.loop(0, K, step=2 * FB)
            def _(g0):
                da = [pltpu.async_copy(table_h.at[si_v.at[g0 + f]],
                                       bufs[f], sga) for f in range(FB)]
                db = [pltpu.async_copy(table_h.at[si_v.at[g0 + FB + f]],
                                       bufs[FB + f], sgb) for f in range(FB)]
                for d in da:
                    d.wait()
                sa = [pltpu.async_copy(bufs[f], acc.at[di_v.at[g0 + f]],
                                       ssa, add=True) for f in range(FB)]
                for d in db:
                    d.wait()
                sb = [pltpu.async_copy(bufs[FB + f],
                                       acc.at[di_v.at[g0 + FB + f]],
                                       ssb, add=True) for f in range(FB)]
                for d in sa:
                    d.wait()
                for d in sb:
                    d.wait()

            plsc.subcore_barrier()
            pltpu.sync_copy(acc.at[pl.ds(tb, STRIPE)],
                            out_h.at[b, c, pl.ds(tb, STRIPE)])
            plsc.subcore_barrier()

    return agg_kernel(src8, dst3, table, zb_hbm)


def _sc_agg16(src3, dst3, table, zb_hbm):
    @functools.partial(
        pl.kernel,
        out_type=jax.ShapeDtypeStruct((2, NPAD, 16), jnp.float32),
        mesh=_mesh,
        compiler_params=pltpu.CompilerParams(use_tc_tiling_on_sc=False),
        scratch_types=[
            pltpu.VMEM((K, CH), jnp.int32),
            pltpu.VMEM((K, CH), jnp.int32),
            [pltpu.VMEM((CH, 16), jnp.float32) for _ in range(2 * FB)],
            pltpu.VMEM((ZCH, 16), jnp.float32),
            pltpu.VMEM_SHARED((NPAD, 16), jnp.float32),
            pltpu.SemaphoreType.DMA,
            pltpu.SemaphoreType.DMA,
            pltpu.SemaphoreType.DMA,
            pltpu.SemaphoreType.DMA,
        ],
    )
    def agg_kernel(src_h, dst_h, table_h, zb_h, out_h, si_v, di_v, bufs, zv,
                   acc, sga, sgb, ssa, ssb):
        c = lax.axis_index("c")
        s = lax.axis_index("s")
        wid = c * NS + s
        tb = s * STRIPE
        pltpu.sync_copy(zb_h, zv)
        pltpu.sync_copy(dst_h.at[wid], di_v)
        pltpu.sync_copy(src_h.at[wid], si_v)
        zd = [pltpu.async_copy(zv, acc.at[pl.ds(tb + j * ZCH, ZCH)], sga)
              for j in range(NZB)]
        for d in zd:
            d.wait()
        plsc.subcore_barrier()

        @pl.loop(0, K, step=2 * FB)
        def _(g0):
            da = [pltpu.async_copy(table_h.at[si_v.at[g0 + f]],
                                   bufs[f], sga) for f in range(FB)]
            db = [pltpu.async_copy(table_h.at[si_v.at[g0 + FB + f]],
                                   bufs[FB + f], sgb) for f in range(FB)]
            for d in da:
                d.wait()
            sa = [pltpu.async_copy(bufs[f], acc.at[di_v.at[g0 + f]],
                                   ssa, add=True) for f in range(FB)]
            for d in db:
                d.wait()
            sb = [pltpu.async_copy(bufs[FB + f],
                                   acc.at[di_v.at[g0 + FB + f]],
                                   ssb, add=True) for f in range(FB)]
            for d in sa:
                d.wait()
            for d in sb:
                d.wait()

        plsc.subcore_barrier()
        sl = pl.ds(tb, STRIPE)
        pltpu.sync_copy(acc.at[sl], out_h.at[c, sl])

    return agg_kernel(src3, dst3, table, zb_hbm)


_BLK = 256


def _tc_sage(x, parts, d0, d1, Ws, Wn, bias, M):
    def body(x_ref, p_ref, d0_ref, d1_ref, ws_ref, wn_ref, b_ref,
             m_ref, o_ref):
        inv = 1.0 / jnp.maximum(d0_ref[:, 0:1] + d1_ref[:, 0:1], 1.0)
        h = jnp.dot(x_ref[...], ws_ref[...],
                    preferred_element_type=jnp.float32,
                    precision=lax.Precision.HIGHEST) + b_ref[...]
        wn = wn_ref[...]
        p = p_ref[...]
        for b in range(NB):
            hb = (p[b, 0] + p[b, 1]) * inv
            h = h + jnp.dot(hb, wn[BW * b:BW * (b + 1), :],
                            preferred_element_type=jnp.float32,
                            precision=lax.Precision.HIGHEST)
        h = jnp.maximum(h, 0.0)
        o_ref[...] = jnp.dot(h, m_ref[...], preferred_element_type=jnp.float32,
                             precision=lax.Precision.HIGHEST)

    return pl.pallas_call(
        body,
        grid=(NPAD // _BLK,),
        in_specs=[
            pl.BlockSpec((_BLK, 128), lambda i: (i, 0)),
            pl.BlockSpec((NB, 2, _BLK, BW), lambda i: (0, 0, i, 0)),
            pl.BlockSpec((_BLK, 16), lambda i: (i, 0)),
            pl.BlockSpec((_BLK, 16), lambda i: (i, 0)),
            pl.BlockSpec((128, 128), lambda i: (0, 0)),
            pl.BlockSpec((128, 128), lambda i: (0, 0)),
            pl.BlockSpec((1, 128), lambda i: (0, 0)),
            pl.BlockSpec((128, 16), lambda i: (0, 0)),
        ],
        out_specs=pl.BlockSpec((_BLK, 16), lambda i: (i, 0)),
        out_shape=jax.ShapeDtypeStruct((NPAD, 16), jnp.float32),
    )(x, parts, d0, d1, Ws, Wn, bias, M)


def _tc_final(happ, p0, p1, d0, d1, cvec):
    def body(h_ref, p0_ref, p1_ref, d0_ref, d1_ref, c_ref, o_ref):
        deg = d0_ref[:, 0:1] + d1_ref[:, 0:1]
        agg = (p0_ref[...] + p1_ref[...]) / jnp.maximum(deg, 1.0)
        o_ref[...] = h_ref[...] + agg + c_ref[...]

    return pl.pallas_call(
        body,
        grid=(NPAD // 1024,),
        in_specs=[
            pl.BlockSpec((1024, 16), lambda i: (i, 0)),
            pl.BlockSpec((1024, 16), lambda i: (i, 0)),
            pl.BlockSpec((1024, 16), lambda i: (i, 0)),
            pl.BlockSpec((1024, 16), lambda i: (i, 0)),
            pl.BlockSpec((1024, 16), lambda i: (i, 0)),
            pl.BlockSpec((1, 16), lambda i: (0, 0)),
        ],
        out_specs=pl.BlockSpec((1024, 16), lambda i: (i, 0)),
        out_shape=jax.ShapeDtypeStruct((NPAD, 16), jnp.float32),
    )(happ, p0, p1, d0, d1, cvec)


def kernel(edge_index_u2a, edge_index_a2u, emb_user, emb_app,
           W1_self_u2a, W1_neigh_u2a, b1_u2a,
           W1_self_a2u, W1_neigh_a2u, b1_a2u,
           W2_self_u2a, W2_neigh_u2a, b2_u2a,
           W_cls, b_cls):
    su = edge_index_u2a[0].astype(jnp.int32)
    du = edge_index_u2a[1].astype(jnp.int32)
    sa = edge_index_a2u[0].astype(jnp.int32)
    da = edge_index_a2u[1].astype(jnp.int32)

    pad = E_PAD - E
    pad_src = (jnp.arange(pad, dtype=jnp.int32) * 97) % N_USER
    pad_dst = N_APP + (jnp.arange(pad, dtype=jnp.int32) % 128)
    su_p = jnp.concatenate([su, pad_src])
    du_p = jnp.concatenate([du, pad_dst])
    sa_p = jnp.concatenate([sa, pad_src])
    da_p = jnp.concatenate([da, pad_dst])

    dst_u3 = du_p.reshape(NW, K, CH)
    dst_a3 = da_p.reshape(NW, K, CH)
    src_u3 = su_p.reshape(NW, K, CH)
    su8 = jnp.stack([su_p * NB + b for b in range(NB)]).reshape(NB, NW, K, CH)
    sa8 = jnp.stack([sa_p * NB + b for b in range(NB)]).reshape(NB, NW, K, CH)

    ones16 = jnp.ones((CH, 16), jnp.float32)
    zb_s = jnp.zeros((STRIPE, 16), jnp.float32)
    zb_z = jnp.zeros((ZCH, 16), jnp.float32)

    table_u = emb_user.reshape(N_USER * NB, BW)
    table_a = emb_app.reshape(N_APP * NB, BW)

    deg = _sc_degree(dst_u3, dst_a3, ones16, zb_s)
    parts_app = _sc_agg128(su8, dst_u3, table_u, zb_z)
    parts_user = _sc_agg128(sa8, dst_a3, table_a, zb_z)

    A = W2_self_u2a @ W_cls
    Bm = W2_neigh_u2a @ W_cls
    cvec = (b2_u2a @ W_cls + b_cls).reshape(1, OUT)

    xu = jnp.pad(emb_user, ((0, NPAD - N_USER), (0, 0)))
    xa = jnp.pad(emb_app, ((0, NPAD - N_APP), (0, 0)))

    z_user = _tc_sage(xu, parts_user, deg[1, 0], deg[1, 1],
                      W1_self_a2u, W1_neigh_a2u, b1_a2u.reshape(1, HID), Bm)
    happ = _tc_sage(xa, parts_app, deg[0, 0], deg[0, 1],
                    W1_self_u2a, W1_neigh_u2a, b1_u2a.reshape(1, HID), A)

    parts_c = _sc_agg16(src_u3, dst_u3, z_user, zb_z)

    out = _tc_final(happ, parts_c[0], parts_c[1], deg[0, 0], deg[0, 1], cvec)
    return out[:N_APP]


# speedup vs baseline: 5.3517x; 1.5474x over previous
import functools

import jax
import jax.numpy as jnp
from jax import lax
from jax.experimental import pallas as pl
from jax.experimental.pallas import tpu as pltpu
from jax.experimental.pallas import tpu_sc as plsc

N_USER = 50000
N_APP = 50000
E = 300000
IN = 128
HID = 128
OUT = 16

NC = 2
NS = 16
NW = NC * NS
CH = 128
K = 80
E_PAD = NW * K * CH
NPAD = 50176
STRIPE = NPAD // NS
FB = 4
ZCH = 196
NZB = STRIPE // ZCH

_mesh = plsc.VectorSubcoreMesh(core_axis_name="c", subcore_axis_name="s")


def _sc_degree(dst_u, dst_a, ones_hbm, zb_hbm):
    @functools.partial(
        pl.kernel,
        out_type=jax.ShapeDtypeStruct((2, 2, NPAD, 16), jnp.float32),
        mesh=_mesh,
        compiler_params=pltpu.CompilerParams(use_tc_tiling_on_sc=False),
        scratch_types=[
            pltpu.VMEM((K, CH), jnp.int32),
            pltpu.VMEM((CH, 16), jnp.float32),
            pltpu.VMEM((STRIPE, 16), jnp.float32),
            pltpu.VMEM_SHARED((NPAD, 16), jnp.float32),
        ],
    )
    def deg_kernel(du_h, da_h, ones_h, zb_h, out_h, idx_v, ones_v, zv, acc):
        c = lax.axis_index("c")
        s = lax.axis_index("s")
        wid = c * NS + s
        tb = s * STRIPE
        pltpu.sync_copy(ones_h, ones_v)
        pltpu.sync_copy(zb_h, zv)
        for rel in range(2):
            pltpu.sync_copy(zv, acc.at[pl.ds(tb, STRIPE)])
            plsc.subcore_barrier()
            pltpu.sync_copy((du_h if rel == 0 else da_h).at[wid], idx_v)

            @pl.loop(0, K)
            def _(j):
                pltpu.sync_copy(ones_v, acc.at[idx_v.at[j]], add=True)

            plsc.subcore_barrier()
            sl = pl.ds(tb, STRIPE)
            pltpu.sync_copy(acc.at[sl], out_h.at[rel, c, sl])
            plsc.subcore_barrier()

    return deg_kernel(dst_u, dst_a, ones_hbm, zb_hbm)


NB = 8
BW = 16


def _sc_agg128(src3, dst3, table, zb_hbm):
    @functools.partial(
        pl.kernel,
        out_type=jax.ShapeDtypeStruct((2, NPAD, 128), jnp.float32),
        mesh=_mesh,
        compiler_params=pltpu.CompilerParams(use_tc_tiling_on_sc=False),
        scratch_types=[
            pltpu.VMEM((K, CH), jnp.int32),
            pltpu.VMEM((K, CH), jnp.int32),
            [pltpu.VMEM((CH, BW), jnp.float32) for _ in range(2 * FB)],
            pltpu.VMEM((ZCH, BW), jnp.float32),
            pltpu.VMEM_SHARED((NPAD, BW), jnp.float32),
            pltpu.SemaphoreType.DMA,
            pltpu.SemaphoreType.DMA,
            pltpu.SemaphoreType.DMA,
            pltpu.SemaphoreType.DMA,
        ],
    )
    def agg_kernel(src_h, dst_h, table_h, zb_h, out_h, si_v, di_v, bufs, zv,
                   acc, sga, sgb, ssa, ssb):
        c = lax.axis_index("c")
        s = lax.axis_index("s")
        wid = c * NS + s
        tb = s * STRIPE
        pltpu.sync_copy(zb_h, zv)
        pltpu.sync_copy(dst_h.at[wid], di_v)
        pltpu.sync_copy(src_h.at[wid], si_v)
        for b in range(NB):
            if b > 0:
                @pl.loop(0, K)
                def _(j):
                    for c0 in range(0, CH, 16):
                        sl = (j, pl.ds(c0, 16))
                        si_v[sl] = si_v[sl] + 1

            zd = [pltpu.async_copy(zv, acc.at[pl.ds(tb + j * ZCH, ZCH)], sga)
                  for j in range(NZB)]
            for d in zd:
                d.wait()
            plsc.subcore_barrier()

            @pl.loop(0, K, step=2 * FB)
            def _(g0):
                da = [pltpu.async_copy(table_h.at[si_v.at[g0 + f]],
                                       bufs[f], sga) for f in range(FB)]
                db = [pltpu.async_copy(table_h.at[si_v.at[g0 + FB + f]],
                                       bufs[FB + f], sgb) for f in range(FB)]
                for d in da:
                    d.wait()
                sa = [pltpu.async_copy(bufs[f], acc.at[di_v.at[g0 + f]],
                                       ssa, add=True) for f in range(FB)]
                for d in db:
                    d.wait()
                sb = [pltpu.async_copy(bufs[FB + f],
                                       acc.at[di_v.at[g0 + FB + f]],
                                       ssb, add=True) for f in range(FB)]
                for d in sa:
                    d.wait()
                for d in sb:
                    d.wait()

            plsc.subcore_barrier()
            pltpu.sync_copy(acc.at[pl.ds(tb, STRIPE)],
                            out_h.at[c, pl.ds(tb, STRIPE), pl.ds(BW * b, BW)])
            plsc.subcore_barrier()

    return agg_kernel(src3, dst3, table, zb_hbm)


def _sc_agg16(src3, dst3, table, zb_hbm):
    @functools.partial(
        pl.kernel,
        out_type=jax.ShapeDtypeStruct((2, NPAD, 16), jnp.float32),
        mesh=_mesh,
        compiler_params=pltpu.CompilerParams(use_tc_tiling_on_sc=False),
        scratch_types=[
            pltpu.VMEM((K, CH), jnp.int32),
            pltpu.VMEM((K, CH), jnp.int32),
            [pltpu.VMEM((CH, 16), jnp.float32) for _ in range(2 * FB)],
            pltpu.VMEM((ZCH, 16), jnp.float32),
            pltpu.VMEM_SHARED((NPAD, 16), jnp.float32),
            pltpu.SemaphoreType.DMA,
            pltpu.SemaphoreType.DMA,
            pltpu.SemaphoreType.DMA,
            pltpu.SemaphoreType.DMA,
        ],
    )
    def agg_kernel(src_h, dst_h, table_h, zb_h, out_h, si_v, di_v, bufs, zv,
                   acc, sga, sgb, ssa, ssb):
        c = lax.axis_index("c")
        s = lax.axis_index("s")
        wid = c * NS + s
        tb = s * STRIPE
        pltpu.sync_copy(zb_h, zv)
        pltpu.sync_copy(dst_h.at[wid], di_v)
        pltpu.sync_copy(src_h.at[wid], si_v)
        zd = [pltpu.async_copy(zv, acc.at[pl.ds(tb + j * ZCH, ZCH)], sga)
              for j in range(NZB)]
        for d in zd:
            d.wait()
        plsc.subcore_barrier()

        @pl.loop(0, K, step=2 * FB)
        def _(g0):
            da = [pltpu.async_copy(table_h.at[si_v.at[g0 + f]],
                                   bufs[f], sga) for f in range(FB)]
            db = [pltpu.async_copy(table_h.at[si_v.at[g0 + FB + f]],
                                   bufs[FB + f], sgb) for f in range(FB)]
            for d in da:
                d.wait()
            sa = [pltpu.async_copy(bufs[f], acc.at[di_v.at[g0 + f]],
                                   ssa, add=True) for f in range(FB)]
            for d in db:
                d.wait()
            sb = [pltpu.async_copy(bufs[FB + f],
                                   acc.at[di_v.at[g0 + FB + f]],
                                   ssb, add=True) for f in range(FB)]
            for d in sa:
                d.wait()
            for d in sb:
                d.wait()

        plsc.subcore_barrier()
        sl = pl.ds(tb, STRIPE)
        pltpu.sync_copy(acc.at[sl], out_h.at[c, sl])

    return agg_kernel(src3, dst3, table, zb_hbm)


_BLK = 256


def _tc_sage(x, parts, d0, d1, Ws, Wn, bias, M):
    def body(x_ref, p_ref, d0_ref, d1_ref, ws_ref, wn_ref, b_ref,
             m_ref, o_ref):
        inv = 1.0 / jnp.maximum(d0_ref[:, 0:1] + d1_ref[:, 0:1], 1.0)
        p = p_ref[...]
        hn = (p[0] + p[1]) * inv
        h = (jnp.dot(x_ref[...], ws_ref[...],
                     preferred_element_type=jnp.float32)
             + jnp.dot(hn, wn_ref[...], preferred_element_type=jnp.float32)
             + b_ref[...])
        h = jnp.maximum(h, 0.0)
        o_ref[...] = jnp.dot(h, m_ref[...],
                             preferred_element_type=jnp.float32)

    return pl.pallas_call(
        body,
        grid=(NPAD // _BLK,),
        in_specs=[
            pl.BlockSpec((_BLK, 128), lambda i: (i, 0)),
            pl.BlockSpec((2, _BLK, 128), lambda i: (0, i, 0)),
            pl.BlockSpec((_BLK, 16), lambda i: (i, 0)),
            pl.BlockSpec((_BLK, 16), lambda i: (i, 0)),
            pl.BlockSpec((128, 128), lambda i: (0, 0)),
            pl.BlockSpec((128, 128), lambda i: (0, 0)),
            pl.BlockSpec((1, 128), lambda i: (0, 0)),
            pl.BlockSpec((128, 16), lambda i: (0, 0)),
        ],
        out_specs=pl.BlockSpec((_BLK, 16), lambda i: (i, 0)),
        out_shape=jax.ShapeDtypeStruct((NPAD, 16), jnp.float32),
    )(x, parts, d0, d1, Ws, Wn, bias, M)


def _tc_final(happ, p0, p1, d0, d1, cvec):
    def body(h_ref, p0_ref, p1_ref, d0_ref, d1_ref, c_ref, o_ref):
        deg = d0_ref[:, 0:1] + d1_ref[:, 0:1]
        agg = (p0_ref[...] + p1_ref[...]) / jnp.maximum(deg, 1.0)
        o_ref[...] = h_ref[...] + agg + c_ref[...]

    return pl.pallas_call(
        body,
        grid=(NPAD // 1024,),
        in_specs=[
            pl.BlockSpec((1024, 16), lambda i: (i, 0)),
            pl.BlockSpec((1024, 16), lambda i: (i, 0)),
            pl.BlockSpec((1024, 16), lambda i: (i, 0)),
            pl.BlockSpec((1024, 16), lambda i: (i, 0)),
            pl.BlockSpec((1024, 16), lambda i: (i, 0)),
            pl.BlockSpec((1, 16), lambda i: (0, 0)),
        ],
        out_specs=pl.BlockSpec((1024, 16), lambda i: (i, 0)),
        out_shape=jax.ShapeDtypeStruct((NPAD, 16), jnp.float32),
    )(happ, p0, p1, d0, d1, cvec)


def kernel(edge_index_u2a, edge_index_a2u, emb_user, emb_app,
           W1_self_u2a, W1_neigh_u2a, b1_u2a,
           W1_self_a2u, W1_neigh_a2u, b1_a2u,
           W2_self_u2a, W2_neigh_u2a, b2_u2a,
           W_cls, b_cls):
    su = edge_index_u2a[0].astype(jnp.int32)
    du = edge_index_u2a[1].astype(jnp.int32)
    sa = edge_index_a2u[0].astype(jnp.int32)
    da = edge_index_a2u[1].astype(jnp.int32)

    pad = E_PAD - E
    pad_src = (jnp.arange(pad, dtype=jnp.int32) * 97) % N_USER
    pad_dst = N_APP + (jnp.arange(pad, dtype=jnp.int32) % 128)
    su_p = jnp.concatenate([su, pad_src])
    du_p = jnp.concatenate([du, pad_dst])
    sa_p = jnp.concatenate([sa, pad_src])
    da_p = jnp.concatenate([da, pad_dst])

    dst_u3 = du_p.reshape(NW, K, CH)
    dst_a3 = da_p.reshape(NW, K, CH)
    src_u3 = su_p.reshape(NW, K, CH)
    su8 = (su_p * NB).reshape(NW, K, CH)
    sa8 = (sa_p * NB).reshape(NW, K, CH)

    ones16 = jnp.ones((CH, 16), jnp.float32)
    zb_s = jnp.zeros((STRIPE, 16), jnp.float32)
    zb_z = jnp.zeros((ZCH, 16), jnp.float32)

    table_u = emb_user.reshape(N_USER * NB, BW)
    table_a = emb_app.reshape(N_APP * NB, BW)

    deg = _sc_degree(dst_u3, dst_a3, ones16, zb_s)
    parts_app = _sc_agg128(su8, dst_u3, table_u, zb_z)
    parts_user = _sc_agg128(sa8, dst_a3, table_a, zb_z)

    A = W2_self_u2a @ W_cls
    Bm = W2_neigh_u2a @ W_cls
    cvec = (b2_u2a @ W_cls + b_cls).reshape(1, OUT)

    xu = jnp.pad(emb_user, ((0, NPAD - N_USER), (0, 0)))
    xa = jnp.pad(emb_app, ((0, NPAD - N_APP), (0, 0)))

    z_user = _tc_sage(xu, parts_user, deg[1, 0], deg[1, 1],
                      W1_self_a2u, W1_neigh_a2u, b1_a2u.reshape(1, HID), Bm)
    happ = _tc_sage(xa, parts_app, deg[0, 0], deg[0, 1],
                    W1_self_u2a, W1_neigh_u2a, b1_u2a.reshape(1, HID), A)

    parts_c = _sc_agg16(src_u3, dst_u3, z_user, zb_z)

    out = _tc_final(happ, parts_c[0], parts_c[1], deg[0, 0], deg[0, 1], cvec)
    return out[:N_APP]


# speedup vs baseline: 6.7994x; 1.2705x over previous
import functools

import jax
import jax.numpy as jnp
from jax import lax
from jax.experimental import pallas as pl
from jax.experimental.pallas import tpu as pltpu
from jax.experimental.pallas import tpu_sc as plsc

N_USER = 50000
N_APP = 50000
E = 300000
IN = 128
HID = 128
OUT = 16

NC = 2
NS = 16
NW = NC * NS
CH = 128
K = 80
E_PAD = NW * K * CH
NPAD = 50176
STRIPE = NPAD // NS
FB = 8
ZCH = 196
NZB = STRIPE // ZCH

_mesh = plsc.VectorSubcoreMesh(core_axis_name="c", subcore_axis_name="s")


def _sc_degree(dst_u, dst_a, ones_hbm, zb_hbm):
    @functools.partial(
        pl.kernel,
        out_type=jax.ShapeDtypeStruct((2, 2, NPAD, 128), jnp.float32),
        mesh=_mesh,
        compiler_params=pltpu.CompilerParams(use_tc_tiling_on_sc=False),
        scratch_types=[
            pltpu.VMEM((K, CH), jnp.int32),
            pltpu.VMEM((CH, 16), jnp.float32),
            pltpu.VMEM((STRIPE, 16), jnp.float32),
            pltpu.VMEM_SHARED((NPAD, 16), jnp.float32),
        ],
    )
    def deg_kernel(du_h, da_h, ones_h, zb_h, out_h, idx_v, ones_v, zv, acc):
        c = lax.axis_index("c")
        s = lax.axis_index("s")
        wid = c * NS + s
        tb = s * STRIPE
        pltpu.sync_copy(ones_h, ones_v)
        pltpu.sync_copy(zb_h, zv)
        for rel in range(2):
            pltpu.sync_copy(zv, acc.at[pl.ds(tb, STRIPE)])
            plsc.subcore_barrier()
            pltpu.sync_copy((du_h if rel == 0 else da_h).at[wid], idx_v)

            @pl.loop(0, K)
            def _(j):
                pltpu.sync_copy(ones_v, acc.at[idx_v.at[j]], add=True)

            plsc.subcore_barrier()
            sl = pl.ds(tb, STRIPE)
            pltpu.sync_copy(acc.at[sl], out_h.at[rel, c, sl, pl.ds(0, 16)])
            plsc.subcore_barrier()

    return deg_kernel(dst_u, dst_a, ones_hbm, zb_hbm)


NB = 4
BW = 32


def _sc_agg128(src3, dst3, table, zb_hbm):
    @functools.partial(
        pl.kernel,
        out_type=jax.ShapeDtypeStruct((2, NPAD, 128), jnp.bfloat16),
        mesh=_mesh,
        compiler_params=pltpu.CompilerParams(use_tc_tiling_on_sc=False),
        scratch_types=[
            pltpu.VMEM((K, CH), jnp.int32),
            pltpu.VMEM((K, CH), jnp.int32),
            [pltpu.VMEM((CH, BW), jnp.bfloat16) for _ in range(2 * FB)],
            pltpu.VMEM((ZCH, BW), jnp.bfloat16),
            pltpu.VMEM_SHARED((NPAD, BW), jnp.bfloat16),
            pltpu.SemaphoreType.DMA,
            pltpu.SemaphoreType.DMA,
            pltpu.SemaphoreType.DMA,
            pltpu.SemaphoreType.DMA,
        ],
    )
    def agg_kernel(src_h, dst_h, table_h, zb_h, out_h, si_v, di_v, bufs, zv,
                   acc, sga, sgb, ssa, ssb):
        c = lax.axis_index("c")
        s = lax.axis_index("s")
        wid = c * NS + s
        tb = s * STRIPE
        pltpu.sync_copy(zb_h, zv)
        pltpu.sync_copy(dst_h.at[wid], di_v)
        pltpu.sync_copy(src_h.at[wid], si_v)
        for b in range(NB):
            if b > 0:
                @pl.loop(0, K)
                def _(j):
                    for c0 in range(0, CH, 16):
                        sl = (j, pl.ds(c0, 16))
                        si_v[sl] = si_v[sl] + 1

            zd = [pltpu.async_copy(zv, acc.at[pl.ds(tb + j * ZCH, ZCH)], sga)
                  for j in range(NZB)]
            for d in zd:
                d.wait()
            plsc.subcore_barrier()

            @pl.loop(0, K, step=2 * FB)
            def _(g0):
                da = [pltpu.async_copy(table_h.at[si_v.at[g0 + f]],
                                       bufs[f], sga) for f in range(FB)]
                db = [pltpu.async_copy(table_h.at[si_v.at[g0 + FB + f]],
                                       bufs[FB + f], sgb) for f in range(FB)]
                for d in da:
                    d.wait()
                sa = [pltpu.async_copy(bufs[f], acc.at[di_v.at[g0 + f]],
                                       ssa, add=True) for f in range(FB)]
                for d in db:
                    d.wait()
                sb = [pltpu.async_copy(bufs[FB + f],
                                       acc.at[di_v.at[g0 + FB + f]],
                                       ssb, add=True) for f in range(FB)]
                for d in sa:
                    d.wait()
                for d in sb:
                    d.wait()

            plsc.subcore_barrier()
            pltpu.sync_copy(acc.at[pl.ds(tb, STRIPE)],
                            out_h.at[c, pl.ds(tb, STRIPE), pl.ds(BW * b, BW)])
            plsc.subcore_barrier()

    return agg_kernel(src3, dst3, table, zb_hbm)


def _sc_agg16(src3, dst3, table, zb_hbm):
    @functools.partial(
        pl.kernel,
        out_type=jax.ShapeDtypeStruct((2, NPAD, 128), jnp.float32),
        mesh=_mesh,
        compiler_params=pltpu.CompilerParams(use_tc_tiling_on_sc=False),
        scratch_types=[
            pltpu.VMEM((K, CH), jnp.int32),
            pltpu.VMEM((K, CH), jnp.int32),
            [pltpu.VMEM((CH, 16), jnp.float32) for _ in range(2 * FB)],
            pltpu.VMEM((ZCH, 16), jnp.float32),
            pltpu.VMEM_SHARED((NPAD, 16), jnp.float32),
            pltpu.SemaphoreType.DMA,
            pltpu.SemaphoreType.DMA,
            pltpu.SemaphoreType.DMA,
            pltpu.SemaphoreType.DMA,
        ],
    )
    def agg_kernel(src_h, dst_h, table_h, zb_h, out_h, si_v, di_v, bufs, zv,
                   acc, sga, sgb, ssa, ssb):
        c = lax.axis_index("c")
        s = lax.axis_index("s")
        wid = c * NS + s
        tb = s * STRIPE
        pltpu.sync_copy(zb_h, zv)
        pltpu.sync_copy(dst_h.at[wid], di_v)
        pltpu.sync_copy(src_h.at[wid], si_v)
        zd = [pltpu.async_copy(zv, acc.at[pl.ds(tb + j * ZCH, ZCH)], sga)
              for j in range(NZB)]
        for d in zd:
            d.wait()
        plsc.subcore_barrier()

        @pl.loop(0, K, step=2 * FB)
        def _(g0):
            da = [pltpu.async_copy(table_h.at[si_v.at[g0 + f]],
                                   bufs[f], sga) for f in range(FB)]
            db = [pltpu.async_copy(table_h.at[si_v.at[g0 + FB + f]],
                                   bufs[FB + f], sgb) for f in range(FB)]
            for d in da:
                d.wait()
            sa = [pltpu.async_copy(bufs[f], acc.at[di_v.at[g0 + f]],
                                   ssa, add=True) for f in range(FB)]
            for d in db:
                d.wait()
            sb = [pltpu.async_copy(bufs[FB + f],
                                   acc.at[di_v.at[g0 + FB + f]],
                                   ssb, add=True) for f in range(FB)]
            for d in sa:
                d.wait()
            for d in sb:
                d.wait()

        plsc.subcore_barrier()
        sl = pl.ds(tb, STRIPE)
        pltpu.sync_copy(acc.at[sl], out_h.at[c, sl, pl.ds(0, 16)])

    return agg_kernel(src3, dst3, table, zb_hbm)


_BLK = 1024


def _tc_sage(x, parts, d0, d1, Ws, Wn, bias, M):
    def body(x_ref, p_ref, d0_ref, d1_ref, ws_ref, wn_ref, b_ref,
             m_ref, o_ref):
        inv = 1.0 / jnp.maximum(d0_ref[:, 0:1] + d1_ref[:, 0:1], 1.0)
        p = p_ref[...].astype(jnp.float32)
        hn = (p[0] + p[1]) * inv
        h = (jnp.dot(x_ref[...], ws_ref[...],
                     preferred_element_type=jnp.float32)
             + jnp.dot(hn, wn_ref[...], preferred_element_type=jnp.float32)
             + b_ref[...])
        h = jnp.maximum(h, 0.0)
        o_ref[...] = jnp.dot(h, m_ref[...], preferred_element_type=jnp.float32)

    out_spec = pl.BlockSpec((_BLK, 16), lambda i: (i, 0))
    out_shape = jax.ShapeDtypeStruct((NPAD, 16), jnp.float32)
    return pl.pallas_call(
        body,
        grid=(NPAD // _BLK,),
        in_specs=[
            pl.BlockSpec((_BLK, 128), lambda i: (i, 0)),
            pl.BlockSpec((2, _BLK, 128), lambda i: (0, i, 0)),
            pl.BlockSpec((_BLK, 128), lambda i: (i, 0)),
            pl.BlockSpec((_BLK, 128), lambda i: (i, 0)),
            pl.BlockSpec((128, 128), lambda i: (0, 0)),
            pl.BlockSpec((128, 128), lambda i: (0, 0)),
            pl.BlockSpec((1, 128), lambda i: (0, 0)),
            pl.BlockSpec((128, 16), lambda i: (0, 0)),
        ],
        out_specs=out_spec,
        out_shape=out_shape,
    )(x, parts, d0, d1, Ws, Wn, bias, M)


def _tc_final(happ, p0, p1, d0, d1, cvec):
    def body(h_ref, p0_ref, p1_ref, d0_ref, d1_ref, c_ref, o_ref):
        deg = d0_ref[:, 0:1] + d1_ref[:, 0:1]
        agg = (p0_ref[:, 0:16] + p1_ref[:, 0:16]) / jnp.maximum(deg, 1.0)
        o_ref[...] = h_ref[...] + agg + c_ref[...]

    return pl.pallas_call(
        body,
        grid=(N_APP // 2000,),
        in_specs=[
            pl.BlockSpec((2000, 16), lambda i: (i, 0)),
            pl.BlockSpec((2000, 128), lambda i: (i, 0)),
            pl.BlockSpec((2000, 128), lambda i: (i, 0)),
            pl.BlockSpec((2000, 128), lambda i: (i, 0)),
            pl.BlockSpec((2000, 128), lambda i: (i, 0)),
            pl.BlockSpec((1, 16), lambda i: (0, 0)),
        ],
        out_specs=pl.BlockSpec((2000, 16), lambda i: (i, 0)),
        out_shape=jax.ShapeDtypeStruct((N_APP, 16), jnp.float32),
    )(happ, p0, p1, d0, d1, cvec)


def kernel(edge_index_u2a, edge_index_a2u, emb_user, emb_app,
           W1_self_u2a, W1_neigh_u2a, b1_u2a,
           W1_self_a2u, W1_neigh_a2u, b1_a2u,
           W2_self_u2a, W2_neigh_u2a, b2_u2a,
           W_cls, b_cls):
    su = edge_index_u2a[0].astype(jnp.int32)
    du = edge_index_u2a[1].astype(jnp.int32)
    sa = edge_index_a2u[0].astype(jnp.int32)
    da = edge_index_a2u[1].astype(jnp.int32)

    pad = E_PAD - E
    pad_src = (jnp.arange(pad, dtype=jnp.int32) * 97) % N_USER
    pad_dst = N_APP + (jnp.arange(pad, dtype=jnp.int32) % 128)
    su_p = jnp.concatenate([su, pad_src])
    du_p = jnp.concatenate([du, pad_dst])
    sa_p = jnp.concatenate([sa, pad_src])
    da_p = jnp.concatenate([da, pad_dst])

    dst_u3 = du_p.reshape(NW, K, CH)
    dst_a3 = da_p.reshape(NW, K, CH)
    src_u3 = su_p.reshape(NW, K, CH)
    su8 = (su_p * NB).reshape(NW, K, CH)
    sa8 = (sa_p * NB).reshape(NW, K, CH)

    ones16 = jnp.ones((CH, 16), jnp.float32)
    zb_s = jnp.zeros((STRIPE, 16), jnp.float32)
    zb_z = jnp.zeros((ZCH, 16), jnp.float32)

    table_u = emb_user.astype(jnp.bfloat16).reshape(N_USER * NB, BW)
    table_a = emb_app.astype(jnp.bfloat16).reshape(N_APP * NB, BW)
    zb_b = jnp.zeros((ZCH, BW), jnp.bfloat16)

    parts_user = _sc_agg128(sa8, dst_a3, table_a, zb_b)
    parts_app = _sc_agg128(su8, dst_u3, table_u, zb_b)
    deg = _sc_degree(dst_u3, dst_a3, ones16, zb_s)

    A = W2_self_u2a @ W_cls
    Bm = W2_neigh_u2a @ W_cls
    cvec = (b2_u2a @ W_cls + b_cls).reshape(1, OUT)

    z_user = _tc_sage(emb_user, parts_user, deg[1, 0], deg[1, 1],
                      W1_self_a2u, W1_neigh_a2u, b1_a2u.reshape(1, HID), Bm)
    happ = _tc_sage(emb_app, parts_app, deg[0, 0], deg[0, 1],
                    W1_self_u2a, W1_neigh_u2a, b1_u2a.reshape(1, HID), A)

    parts_c = _sc_agg16(src_u3, dst_u3, z_user, zb_z)

    return _tc_final(happ, parts_c[0], parts_c[1], deg[0, 0], deg[0, 1], cvec)
